# baseline (speedup 1.0000x reference)
import functools

import jax
import jax.numpy as jnp
import numpy as np
from jax import lax
from jax.experimental import pallas as pl
from jax.experimental.pallas import tpu as pltpu
from jax.experimental.pallas import tpu_sc as plsc


@functools.lru_cache(maxsize=None)
def _make_gather(C: int, T: int, H: int, W: int):
    n_slow = T // 4
    lin = [int(v) for v in np.linspace(0.0, T - 1, n_slow).astype(np.int32)]
    assert lin == [(t * (T - 1)) // (n_slow - 1) for t in range(n_slow)]

    info = plsc.get_sparse_core_info()
    n_workers = info.num_cores * info.num_subcores

    n_chunks = C * n_slow * 4
    assert n_chunks % n_workers == 0
    per_w = n_chunks // n_workers
    hq = H // 4
    assert hq % 8 == 0

    mesh = plsc.VectorSubcoreMesh(core_axis_name="c", subcore_axis_name="s")

    @functools.partial(
        pl.kernel,
        out_type=jax.ShapeDtypeStruct((C, n_slow, H, W), jnp.float32),
        mesh=mesh,
        scratch_types=(
            [pltpu.VMEM((per_w * hq, W), jnp.float32)]
            + [pltpu.SemaphoreType.DMA] * (2 * per_w)
        ),
    )
    def gather_frames(src_hbm, tok_hbm, out_hbm, buf, *sems):
        del tok_hbm
        gsem, ssem = sems[:per_w], sems[per_w:]
        wid = lax.axis_index("s") * info.num_cores + lax.axis_index("c")

        def chunk_coords(j):
            k = per_w * wid + j
            r = k // 4
            q = k % 4
            cc = r // n_slow
            ts = r % n_slow
            st = (ts * (T - 1)) // (n_slow - 1)
            return cc, ts, st, q

        handles = []
        for j in range(per_w):
            cc, ts, st, q = chunk_coords(j)
            handles.append(pltpu.async_copy(
                src_hbm.at[cc, st, pl.ds(q * hq, hq), :],
                buf.at[pl.ds(j * hq, hq), :],
                gsem[j]))
        out_handles = []
        for j in range(per_w):
            handles[j].wait()
            cc, ts, st, q = chunk_coords(j)
            out_handles.append(pltpu.async_copy(
                buf.at[pl.ds(j * hq, hq), :],
                out_hbm.at[cc, ts, pl.ds(q * hq, hq), :],
                ssem[j]))
        for h in out_handles:
            h.wait()

    return gather_frames


def _copy_body(src_ref, out_ref):
    out_ref[...] = src_ref[...]


@functools.lru_cache(maxsize=None)
def _make_fast_copy_a(C: int, T: int, H: int, W: int, ta: int):
    def body(src_ref, out_ref):
        out_ref[...] = src_ref[...]

    return pl.pallas_call(
        body,
        grid=(C,),
        in_specs=[pl.BlockSpec((1, ta, H, W), lambda c: (c, 0, 0, 0))],
        out_specs=pl.BlockSpec((1, ta, H, W), lambda c: (c, 0, 0, 0)),
        out_shape=jax.ShapeDtypeStruct((C, T, H, W), jnp.float32),
    )


@functools.lru_cache(maxsize=None)
def _make_fast_copy_b(C: int, T: int, H: int, W: int, ta: int):
    tb = T - ta

    def body(src_ref, acc_ref, out_ref):
        del acc_ref
        out_ref[...] = src_ref[...]

    return pl.pallas_call(
        body,
        grid=(C,),
        in_specs=[
            pl.BlockSpec((1, tb, H, W), lambda c: (c, 1, 0, 0)),
            pl.BlockSpec(memory_space=pl.ANY),
        ],
        out_specs=pl.BlockSpec((1, tb, H, W), lambda c: (c, 1, 0, 0)),
        out_shape=jax.ShapeDtypeStruct((C, T, H, W), jnp.float32),
        input_output_aliases={1: 0},
    )


def kernel(frames):
    C, T, H, W = frames.shape
    ta = T // 2
    fast_a = _make_fast_copy_a(C, T, H, W, ta)(frames)
    slow = _make_gather(C, T, H, W)(frames, fast_a)
    fast = _make_fast_copy_b(C, T, H, W, ta)(frames, fast_a)
    return (slow, fast)


# speedup vs baseline: 1.1217x; 1.1217x over previous
import functools

import jax
import jax.numpy as jnp
import numpy as np
from jax import lax
from jax.experimental import pallas as pl
from jax.experimental.pallas import tpu as pltpu
from jax.experimental.pallas import tpu_sc as plsc


@functools.lru_cache(maxsize=None)
def _make_gather(C: int, T: int, H: int, W: int):
    n_slow = T // 4
    lin = [int(v) for v in np.linspace(0.0, T - 1, n_slow).astype(np.int32)]
    assert lin == [(t * (T - 1)) // (n_slow - 1) for t in range(n_slow)]

    info = plsc.get_sparse_core_info()
    n_workers = info.num_cores * info.num_subcores

    n_chunks = C * n_slow * 4
    assert n_chunks % n_workers == 0
    per_w = n_chunks // n_workers
    hq = H // 4
    assert hq % 8 == 0

    mesh = plsc.VectorSubcoreMesh(core_axis_name="c", subcore_axis_name="s")

    @functools.partial(
        pl.kernel,
        out_type=jax.ShapeDtypeStruct((C, n_slow, H, W), jnp.float32),
        mesh=mesh,
        scratch_types=(
            [pltpu.VMEM((per_w * hq, W), jnp.float32)]
            + [pltpu.SemaphoreType.DMA] * (2 * per_w)
        ),
    )
    def gather_frames(src_hbm, tok_hbm, out_hbm, buf, *sems):
        del tok_hbm
        gsem, ssem = sems[:per_w], sems[per_w:]
        wid = lax.axis_index("s") * info.num_cores + lax.axis_index("c")

        def chunk_coords(j):
            k = per_w * wid + j
            r = k // 4
            q = k % 4
            cc = r // n_slow
            ts = r % n_slow
            st = (ts * (T - 1)) // (n_slow - 1)
            return cc, ts, st, q

        handles = []
        for j in range(per_w):
            cc, ts, st, q = chunk_coords(j)
            handles.append(pltpu.async_copy(
                src_hbm.at[cc, st, pl.ds(q * hq, hq), :],
                buf.at[pl.ds(j * hq, hq), :],
                gsem[j]))
        out_handles = []
        for j in range(per_w):
            handles[j].wait()
            cc, ts, st, q = chunk_coords(j)
            out_handles.append(pltpu.async_copy(
                buf.at[pl.ds(j * hq, hq), :],
                out_hbm.at[cc, ts, pl.ds(q * hq, hq), :],
                ssem[j]))
        for h in out_handles:
            h.wait()

    return gather_frames


def _copy_body(src_ref, out_ref):
    out_ref[...] = src_ref[...]


@functools.lru_cache(maxsize=None)
def _make_fast_copy_a(C: int, T: int, H: int, W: int, ta: int):
    def body(src_ref, out_ref):
        out_ref[...] = src_ref[...]

    return pl.pallas_call(
        body,
        grid=(C,),
        in_specs=[pl.BlockSpec((1, ta, H, W), lambda c: (c, 0, 0, 0))],
        out_specs=pl.BlockSpec((1, ta, H, W), lambda c: (c, 0, 0, 0)),
        out_shape=jax.ShapeDtypeStruct((C, T, H, W), jnp.float32),
    )


@functools.lru_cache(maxsize=None)
def _make_fast_copy_b(C: int, T: int, H: int, W: int, ta: int):
    tb = T - ta

    def body(src_ref, acc_ref, out_ref):
        del acc_ref
        out_ref[...] = src_ref[...]

    return pl.pallas_call(
        body,
        grid=(C,),
        in_specs=[
            pl.BlockSpec((1, tb, H, W), lambda c: (c, 1, 0, 0)),
            pl.BlockSpec(memory_space=pl.ANY),
        ],
        out_specs=pl.BlockSpec((1, tb, H, W), lambda c: (c, 1, 0, 0)),
        out_shape=jax.ShapeDtypeStruct((C, T, H, W), jnp.float32),
        input_output_aliases={1: 0},
    )


@functools.lru_cache(maxsize=None)
def _make_fast_copy(C: int, T: int, H: int, W: int):
    tb = 16
    assert T % tb == 0

    return pl.pallas_call(
        _copy_body,
        grid=(C, T // tb),
        in_specs=[pl.BlockSpec((1, tb, H, W), lambda c, t: (c, t, 0, 0))],
        out_specs=pl.BlockSpec((1, tb, H, W), lambda c, t: (c, t, 0, 0)),
        out_shape=jax.ShapeDtypeStruct((C, T, H, W), jnp.float32),
    )


@functools.lru_cache(maxsize=None)
def _make_gather_scs(C: int, T: int, H: int, W: int):
    n_slow = T // 4
    n_sel = C * n_slow
    per_core = n_sel // 2
    mesh = plsc.ScalarSubcoreMesh(axis_name="c", num_cores=2)

    @functools.partial(
        pl.kernel,
        out_type=jax.ShapeDtypeStruct((C, n_slow, H, W), jnp.float32),
        mesh=mesh,
        scratch_types=(
            [pltpu.VMEM_SHARED((per_core, H, W), jnp.float32)]
            + [pltpu.SemaphoreType.DMA] * (2 * per_core)
        ),
    )
    def gather_frames(src_hbm, out_hbm, buf, *sems):
        gsem, ssem = sems[:per_core], sems[per_core:]
        core = lax.axis_index("c")
        for half in range(2):
            @pl.when(core == half)
            def _copy(half=half):
                handles = []
                for i in range(per_core):
                    k = half * per_core + i
                    cc, ts = k // n_slow, k % n_slow
                    st = (ts * (T - 1)) // (n_slow - 1)
                    handles.append(pltpu.async_copy(
                        src_hbm.at[cc, st], buf.at[i], gsem[i]))
                out_handles = []
                for i in range(per_core):
                    handles[i].wait()
                    k = half * per_core + i
                    cc, ts = k // n_slow, k % n_slow
                    out_handles.append(pltpu.async_copy(
                        buf.at[i], out_hbm.at[cc, ts], ssem[i]))
                for h in out_handles:
                    h.wait()

    return gather_frames


def kernel(frames):
    C, T, H, W = frames.shape
    slow = _make_gather_scs(C, T, H, W)(frames)
    fast = _make_fast_copy(C, T, H, W)(frames)
    return (slow, fast)


# speedup vs baseline: 1.1643x; 1.0380x over previous
import functools

import jax
import jax.numpy as jnp
import numpy as np
from jax import lax
from jax.experimental import pallas as pl
from jax.experimental.pallas import tpu as pltpu
from jax.experimental.pallas import tpu_sc as plsc


@functools.lru_cache(maxsize=None)
def _make_gather(C: int, T: int, H: int, W: int):
    n_slow = T // 4
    lin = [int(v) for v in np.linspace(0.0, T - 1, n_slow).astype(np.int32)]
    assert lin == [(t * (T - 1)) // (n_slow - 1) for t in range(n_slow)]

    info = plsc.get_sparse_core_info()
    n_workers = info.num_cores * info.num_subcores

    n_chunks = C * n_slow * 4
    assert n_chunks % n_workers == 0
    per_w = n_chunks // n_workers
    hq = H // 4
    assert hq % 8 == 0

    mesh = plsc.VectorSubcoreMesh(core_axis_name="c", subcore_axis_name="s")

    @functools.partial(
        pl.kernel,
        out_type=jax.ShapeDtypeStruct((C, n_slow, H, W), jnp.float32),
        mesh=mesh,
        scratch_types=(
            [pltpu.VMEM((per_w * hq, W), jnp.float32)]
            + [pltpu.SemaphoreType.DMA] * (2 * per_w)
        ),
    )
    def gather_frames(src_hbm, out_hbm, buf, *sems):
        gsem, ssem = sems[:per_w], sems[per_w:]
        wid = lax.axis_index("s") * info.num_cores + lax.axis_index("c")

        def chunk_coords(j):
            k = per_w * wid + j
            r = k // 4
            q = k % 4
            cc = r // n_slow
            ts = r % n_slow
            st = (ts * (T - 1)) // (n_slow - 1)
            return cc, ts, st, q

        handles = []
        for j in range(per_w):
            cc, ts, st, q = chunk_coords(j)
            handles.append(pltpu.async_copy(
                src_hbm.at[cc, st, pl.ds(q * hq, hq), :],
                buf.at[pl.ds(j * hq, hq), :],
                gsem[j]))
        out_handles = []
        for j in range(per_w):
            handles[j].wait()
            cc, ts, st, q = chunk_coords(j)
            out_handles.append(pltpu.async_copy(
                buf.at[pl.ds(j * hq, hq), :],
                out_hbm.at[cc, ts, pl.ds(q * hq, hq), :],
                ssem[j]))
        for h in out_handles:
            h.wait()

    return gather_frames


def _copy_body(src_ref, out_ref):
    out_ref[...] = src_ref[...]


@functools.lru_cache(maxsize=None)
def _make_fast_copy_a(C: int, T: int, H: int, W: int, ta: int):
    def body(src_ref, out_ref):
        out_ref[...] = src_ref[...]

    return pl.pallas_call(
        body,
        grid=(C,),
        in_specs=[pl.BlockSpec((1, ta, H, W), lambda c: (c, 0, 0, 0))],
        out_specs=pl.BlockSpec((1, ta, H, W), lambda c: (c, 0, 0, 0)),
        out_shape=jax.ShapeDtypeStruct((C, T, H, W), jnp.float32),
    )


@functools.lru_cache(maxsize=None)
def _make_fast_copy_b(C: int, T: int, H: int, W: int, ta: int):
    tb = T - ta

    def body(src_ref, acc_ref, out_ref):
        del acc_ref
        out_ref[...] = src_ref[...]

    return pl.pallas_call(
        body,
        grid=(C,),
        in_specs=[
            pl.BlockSpec((1, tb, H, W), lambda c: (c, 1, 0, 0)),
            pl.BlockSpec(memory_space=pl.ANY),
        ],
        out_specs=pl.BlockSpec((1, tb, H, W), lambda c: (c, 1, 0, 0)),
        out_shape=jax.ShapeDtypeStruct((C, T, H, W), jnp.float32),
        input_output_aliases={1: 0},
    )


@functools.lru_cache(maxsize=None)
def _make_fast_copy(C: int, T: int, H: int, W: int):
    tb = 32
    assert T % tb == 0

    return pl.pallas_call(
        _copy_body,
        grid=(C, T // tb),
        in_specs=[pl.BlockSpec((1, tb, H, W), lambda c, t: (c, t, 0, 0))],
        out_specs=pl.BlockSpec((1, tb, H, W), lambda c, t: (c, t, 0, 0)),
        out_shape=jax.ShapeDtypeStruct((C, T, H, W), jnp.float32),
    )


@functools.lru_cache(maxsize=None)
def _make_gather_scs(C: int, T: int, H: int, W: int):
    n_slow = T // 4
    n_sel = C * n_slow
    per_core = n_sel // 2
    mesh = plsc.ScalarSubcoreMesh(axis_name="c", num_cores=2)

    @functools.partial(
        pl.kernel,
        out_type=jax.ShapeDtypeStruct((C, n_slow, H, W), jnp.float32),
        mesh=mesh,
        scratch_types=(
            [pltpu.VMEM_SHARED((per_core, H, W), jnp.float32)]
            + [pltpu.SemaphoreType.DMA] * (2 * per_core)
        ),
    )
    def gather_frames(src_hbm, out_hbm, buf, *sems):
        gsem, ssem = sems[:per_core], sems[per_core:]
        core = lax.axis_index("c")
        for half in range(2):
            @pl.when(core == half)
            def _copy(half=half):
                handles = []
                for i in range(per_core):
                    k = half * per_core + i
                    cc, ts = k // n_slow, k % n_slow
                    st = (ts * (T - 1)) // (n_slow - 1)
                    handles.append(pltpu.async_copy(
                        src_hbm.at[cc, st], buf.at[i], gsem[i]))
                out_handles = []
                for i in range(per_core):
                    handles[i].wait()
                    k = half * per_core + i
                    cc, ts = k // n_slow, k % n_slow
                    out_handles.append(pltpu.async_copy(
                        buf.at[i], out_hbm.at[cc, ts], ssem[i]))
                for h in out_handles:
                    h.wait()

    return gather_frames


def kernel(frames):
    C, T, H, W = frames.shape
    slow = _make_gather(C, T, H, W)(frames)
    fast = _make_fast_copy(C, T, H, W)(frames)
    return (slow, fast)


# speedup vs baseline: 1.2153x; 1.0438x over previous
import functools

import jax
import jax.numpy as jnp
import numpy as np
from jax import lax
from jax.experimental import pallas as pl
from jax.experimental.pallas import tpu as pltpu
from jax.experimental.pallas import tpu_sc as plsc


@functools.lru_cache(maxsize=None)
def _make_gather(C: int, T: int, H: int, W: int):
    n_slow = T // 4
    lin = [int(v) for v in np.linspace(0.0, T - 1, n_slow).astype(np.int32)]
    assert lin == [(t * (T - 1)) // (n_slow - 1) for t in range(n_slow)]

    info = plsc.get_sparse_core_info()
    n_workers = info.num_subcores

    n_chunks = C * n_slow * 4
    assert n_chunks % n_workers == 0
    per_w = n_chunks // n_workers
    hq = H // 4
    assert hq % 8 == 0

    mesh = plsc.VectorSubcoreMesh(
        core_axis_name="c", subcore_axis_name="s", num_cores=1)

    @functools.partial(
        pl.kernel,
        out_type=jax.ShapeDtypeStruct((C, n_slow, H, W), jnp.float32),
        mesh=mesh,
        scratch_types=(
            [pltpu.VMEM((per_w * hq, W), jnp.float32)]
            + [pltpu.SemaphoreType.DMA] * (2 * per_w)
        ),
    )
    def gather_frames(src_hbm, out_hbm, buf, *sems):
        gsem, ssem = sems[:per_w], sems[per_w:]
        wid = lax.axis_index("s")

        def chunk_coords(j):
            k = per_w * wid + j
            r = k // 4
            q = k % 4
            cc = r // n_slow
            ts = r % n_slow
            st = (ts * (T - 1)) // (n_slow - 1)
            return cc, ts, st, q

        handles = []
        for j in range(per_w):
            cc, ts, st, q = chunk_coords(j)
            handles.append(pltpu.async_copy(
                src_hbm.at[cc, st, pl.ds(q * hq, hq), :],
                buf.at[pl.ds(j * hq, hq), :],
                gsem[j]))
        out_handles = []
        for j in range(per_w):
            handles[j].wait()
            cc, ts, st, q = chunk_coords(j)
            out_handles.append(pltpu.async_copy(
                buf.at[pl.ds(j * hq, hq), :],
                out_hbm.at[cc, ts, pl.ds(q * hq, hq), :],
                ssem[j]))
        for h in out_handles:
            h.wait()

    return gather_frames


def _copy_body(src_ref, out_ref):
    out_ref[...] = src_ref[...]


@functools.lru_cache(maxsize=None)
def _make_fast_copy_a(C: int, T: int, H: int, W: int, ta: int):
    def body(src_ref, out_ref):
        out_ref[...] = src_ref[...]

    return pl.pallas_call(
        body,
        grid=(C,),
        in_specs=[pl.BlockSpec((1, ta, H, W), lambda c: (c, 0, 0, 0))],
        out_specs=pl.BlockSpec((1, ta, H, W), lambda c: (c, 0, 0, 0)),
        out_shape=jax.ShapeDtypeStruct((C, T, H, W), jnp.float32),
    )


@functools.lru_cache(maxsize=None)
def _make_fast_copy_b(C: int, T: int, H: int, W: int, ta: int):
    tb = T - ta

    def body(src_ref, acc_ref, out_ref):
        del acc_ref
        out_ref[...] = src_ref[...]

    return pl.pallas_call(
        body,
        grid=(C,),
        in_specs=[
            pl.BlockSpec((1, tb, H, W), lambda c: (c, 1, 0, 0)),
            pl.BlockSpec(memory_space=pl.ANY),
        ],
        out_specs=pl.BlockSpec((1, tb, H, W), lambda c: (c, 1, 0, 0)),
        out_shape=jax.ShapeDtypeStruct((C, T, H, W), jnp.float32),
        input_output_aliases={1: 0},
    )


@functools.lru_cache(maxsize=None)
def _make_fast_copy(C: int, T: int, H: int, W: int):
    tb = 32
    assert T % tb == 0

    return pl.pallas_call(
        _copy_body,
        grid=(C, T // tb),
        in_specs=[pl.BlockSpec((1, tb, H, W), lambda c, t: (c, t, 0, 0))],
        out_specs=pl.BlockSpec((1, tb, H, W), lambda c, t: (c, t, 0, 0)),
        out_shape=jax.ShapeDtypeStruct((C, T, H, W), jnp.float32),
    )


@functools.lru_cache(maxsize=None)
def _make_gather_scs(C: int, T: int, H: int, W: int):
    n_slow = T // 4
    n_sel = C * n_slow
    per_core = n_sel // 2
    mesh = plsc.ScalarSubcoreMesh(axis_name="c", num_cores=2)

    @functools.partial(
        pl.kernel,
        out_type=jax.ShapeDtypeStruct((C, n_slow, H, W), jnp.float32),
        mesh=mesh,
        scratch_types=(
            [pltpu.VMEM_SHARED((per_core, H, W), jnp.float32)]
            + [pltpu.SemaphoreType.DMA] * (2 * per_core)
        ),
    )
    def gather_frames(src_hbm, out_hbm, buf, *sems):
        gsem, ssem = sems[:per_core], sems[per_core:]
        core = lax.axis_index("c")
        for half in range(2):
            @pl.when(core == half)
            def _copy(half=half):
                handles = []
                for i in range(per_core):
                    k = half * per_core + i
                    cc, ts = k // n_slow, k % n_slow
                    st = (ts * (T - 1)) // (n_slow - 1)
                    handles.append(pltpu.async_copy(
                        src_hbm.at[cc, st], buf.at[i], gsem[i]))
                out_handles = []
                for i in range(per_core):
                    handles[i].wait()
                    k = half * per_core + i
                    cc, ts = k // n_slow, k % n_slow
                    out_handles.append(pltpu.async_copy(
                        buf.at[i], out_hbm.at[cc, ts], ssem[i]))
                for h in out_handles:
                    h.wait()

    return gather_frames


def kernel(frames):
    C, T, H, W = frames.shape
    slow = _make_gather(C, T, H, W)(frames)
    fast = _make_fast_copy(C, T, H, W)(frames)
    return (slow, fast)
